# baseline (speedup 1.0000x reference)
import functools

import jax
import jax.numpy as jnp
from jax.experimental import pallas as pl
from jax.experimental.pallas import tpu as pltpu

_E = 64
_K = 8
_BT = 1024
_NK = 2


def _gate_kernel(x_ref, w_ref, idx_ref, score_ref, acc_ref):
    k = pl.program_id(1)
    part = jax.lax.dot_general(
        w_ref[...], x_ref[...], (((1,), (1,)), ((), ())),
        preferred_element_type=jnp.float32,
    )

    @pl.when(k == 0)
    def _init():
        acc_ref[...] = part

    @pl.when(k > 0)
    def _acc():
        acc_ref[...] += part

    @pl.when(k == _NK - 1)
    def _finish():
        logits = acc_ref[...]
        bt = logits.shape[1]
        iota = jax.lax.broadcasted_iota(jnp.int32, (_E, bt), 0)
        idx_rows = []
        val_rows = []
        work = logits
        for _ in range(_K):
            vmax = jnp.max(work, axis=0, keepdims=True)
            cand = jnp.where(work == vmax, iota, _E)
            imin = jnp.min(cand, axis=0, keepdims=True)
            idx_rows.append(imin)
            val_rows.append(vmax)
            work = jnp.where(iota == imin, -1e30, work)
        topk_idx = jnp.concatenate(idx_rows, axis=0)
        topk_logit = jnp.concatenate(val_rows, axis=0)
        e = jnp.exp(topk_logit - topk_logit[0:1])
        denom = jnp.sum(e, axis=0, keepdims=True)
        idx_ref[...] = topk_idx
        score_ref[...] = e / denom


@functools.partial(jax.jit, static_argnames=())
def kernel(hidden_states, weight):
    b, s, d = hidden_states.shape
    t = b * s
    dk = d // _NK
    x = hidden_states.reshape(t, d)
    grid = (t // _BT, _NK)
    idx_t, scores_t = pl.pallas_call(
        _gate_kernel,
        grid=grid,
        in_specs=[
            pl.BlockSpec((_BT, dk), lambda i, k: (i, k)),
            pl.BlockSpec((_E, dk), lambda i, k: (0, k)),
        ],
        out_specs=[
            pl.BlockSpec((_K, _BT), lambda i, k: (0, i)),
            pl.BlockSpec((_K, _BT), lambda i, k: (0, i)),
        ],
        out_shape=[
            jax.ShapeDtypeStruct((_K, t), jnp.int32),
            jax.ShapeDtypeStruct((_K, t), jnp.float32),
        ],
        scratch_shapes=[pltpu.VMEM((_E, _BT), jnp.float32)],
        compiler_params=pltpu.CompilerParams(
            dimension_semantics=("arbitrary", "arbitrary"),
        ),
    )(x, weight)
    aux_loss = jnp.zeros((), dtype=jnp.float32)
    return (idx_t.T, scores_t.T, aux_loss)


# speedup vs baseline: 1.1478x; 1.1478x over previous
import functools

import jax
import jax.numpy as jnp
from jax.experimental import pallas as pl
from jax.experimental.pallas import tpu as pltpu

_E = 64
_K = 8
_BT = 1024


def _topk_store(logits, idx_ref, score_ref):
    bt = logits.shape[1]
    iota = jax.lax.broadcasted_iota(jnp.int32, (_E, bt), 0)
    idx_rows = []
    val_rows = []
    work = logits
    for _ in range(_K):
        vmax = jnp.max(work, axis=0, keepdims=True)
        cand = jnp.where(work == vmax, iota, _E)
        imin = jnp.min(cand, axis=0, keepdims=True)
        idx_rows.append(imin)
        val_rows.append(vmax)
        work = jnp.where(iota == imin, -1e30, work)
    topk_idx = jnp.concatenate(idx_rows, axis=0)
    topk_logit = jnp.concatenate(val_rows, axis=0)
    e = jnp.exp(topk_logit - topk_logit[0:1])
    denom = jnp.sum(e, axis=0, keepdims=True)
    idx_ref[...] = topk_idx
    score_ref[...] = e / denom


def _gate_kernel(x_ref, w_ref, idx_ref, score_ref, la_ref, lb_ref):
    i = pl.program_id(0)
    n = pl.num_programs(0)
    even = jax.lax.rem(i, 2) == 0

    @pl.when(i < n - 1)
    def _matmul():
        part = jax.lax.dot_general(
            w_ref[...], x_ref[...], (((1,), (1,)), ((), ())),
            preferred_element_type=jnp.float32,
        )

        @pl.when(even)
        def _sa():
            la_ref[...] = part

        @pl.when(jnp.logical_not(even))
        def _sb():
            lb_ref[...] = part

    @pl.when(i > 0)
    def _finish():
        @pl.when(even)
        def _fb():
            _topk_store(lb_ref[...], idx_ref, score_ref)

        @pl.when(jnp.logical_not(even))
        def _fa():
            _topk_store(la_ref[...], idx_ref, score_ref)


@functools.partial(jax.jit, static_argnames=())
def kernel(hidden_states, weight):
    b, s, d = hidden_states.shape
    t = b * s
    x = hidden_states.reshape(t, d)
    nb = t // _BT
    grid = (nb + 1,)
    idx_t, scores_t = pl.pallas_call(
        _gate_kernel,
        grid=grid,
        in_specs=[
            pl.BlockSpec((_BT, d), lambda i: (jnp.minimum(i, nb - 1), 0)),
            pl.BlockSpec((_E, d), lambda i: (0, 0)),
        ],
        out_specs=[
            pl.BlockSpec((_K, _BT), lambda i: (0, jnp.maximum(i - 1, 0))),
            pl.BlockSpec((_K, _BT), lambda i: (0, jnp.maximum(i - 1, 0))),
        ],
        out_shape=[
            jax.ShapeDtypeStruct((_K, t), jnp.int32),
            jax.ShapeDtypeStruct((_K, t), jnp.float32),
        ],
        scratch_shapes=[
            pltpu.VMEM((_E, _BT), jnp.float32),
            pltpu.VMEM((_E, _BT), jnp.float32),
        ],
    )(x, weight)
    aux_loss = jnp.zeros((), dtype=jnp.float32)
    return (idx_t.T, scores_t.T, aux_loss)


# speedup vs baseline: 1.1590x; 1.0097x over previous
import functools

import jax
import jax.numpy as jnp
from jax.experimental import pallas as pl

_E = 64
_K = 8
_BT = 1024


def _gate_kernel(x_ref, w_ref, idx_ref, score_ref):
    x = x_ref[...]
    w = w_ref[...]
    logits = jax.lax.dot_general(
        w, x, (((1,), (1,)), ((), ())), preferred_element_type=jnp.float32
    )
    bt = logits.shape[1]
    iota = jax.lax.broadcasted_iota(jnp.int32, (_E, bt), 0)
    idx_rows = []
    val_rows = []
    work = logits
    for _ in range(_K):
        vmax = jnp.max(work, axis=0, keepdims=True)
        cand = jnp.where(work == vmax, iota, _E)
        imin = jnp.min(cand, axis=0, keepdims=True)
        idx_rows.append(imin)
        val_rows.append(vmax)
        work = jnp.where(iota == imin, -1e30, work)
    topk_idx = jnp.concatenate(idx_rows, axis=0)
    topk_logit = jnp.concatenate(val_rows, axis=0)
    e = jnp.exp(topk_logit - topk_logit[0:1])
    denom = jnp.sum(e, axis=0, keepdims=True)
    idx_ref[...] = topk_idx
    score_ref[...] = e / denom


@functools.partial(jax.jit, static_argnames=())
def kernel(hidden_states, weight):
    b, s, d = hidden_states.shape
    t = b * s
    x = hidden_states.reshape(t, d)
    grid = (t // _BT,)
    idx_t, scores_t = pl.pallas_call(
        _gate_kernel,
        grid=grid,
        in_specs=[
            pl.BlockSpec((_BT, d), lambda i: (i, 0)),
            pl.BlockSpec((_E, d), lambda i: (0, 0)),
        ],
        out_specs=[
            pl.BlockSpec((_K, _BT), lambda i: (0, i)),
            pl.BlockSpec((_K, _BT), lambda i: (0, i)),
        ],
        out_shape=[
            jax.ShapeDtypeStruct((_K, t), jnp.int32),
            jax.ShapeDtypeStruct((_K, t), jnp.float32),
        ],
    )(x, weight)
    aux_loss = jnp.zeros((), dtype=jnp.float32)
    return (idx_t.T, scores_t.T, aux_loss)
